# Initial kernel scaffold; baseline (speedup 1.0000x reference)
import functools

import jax
import jax.numpy as jnp
from jax import lax
from jax.experimental import pallas as pl
from jax.experimental.pallas import tpu as pltpu
from jax.experimental.pallas import tpu_sc as plsc

NC = 2
NS = 16
BATCH = 128


def _make_sc_call(R, D, C, per_tile, N):
    n_chunks_per_core = (R // C) // NC
    mesh = plsc.VectorSubcoreMesh(
        core_axis_name="c", subcore_axis_name="s",
        num_cores=NC, num_subcores=NS)
    zrows = (C + NS) // NS
    wrows = C // NS

    @functools.partial(
        pl.kernel,
        out_type=jax.ShapeDtypeStruct((R, D), jnp.float32),
        mesh=mesh,
        scratch_types=[
            pltpu.VMEM_SHARED((C + NS, 128), jnp.float32),
            pltpu.VMEM((BATCH,), jnp.int32),
            pltpu.VMEM((BATCH,), jnp.int32),
            pltpu.VMEM((BATCH,), jnp.int32),
            pltpu.VMEM((BATCH,), jnp.int32),
            pltpu.VMEM((BATCH,), jnp.int32),
            pltpu.VMEM((BATCH,), jnp.int32),
            pltpu.VMEM((BATCH,), jnp.int32),
            pltpu.VMEM((BATCH, 128), jnp.float32),
            pltpu.VMEM((BATCH, 128), jnp.float32),
            pltpu.SemaphoreType.DMA,
            pltpu.SemaphoreType.DMA,
            pltpu.SemaphoreType.DMA,
        ],
    )
    def sc_call(xf_h, bi_h, si_h, di_h, z_h, out_h,
                acc, biv, siv, div, g0v, g1v, s0v, s1v, st0, st1,
                semi, sem0, sem1):
        c = lax.axis_index("c")
        s = lax.axis_index("s")
        n_batches = per_tile // BATCH

        for p in range(n_chunks_per_core):
            lo = (n_chunks_per_core * c + p) * C
            pltpu.sync_copy(z_h.at[pl.ds(s * zrows, zrows)],
                            acc.at[pl.ds(s * zrows, zrows)])
            plsc.subcore_barrier()

            def body(i, carry):
                base = s * per_tile + i * BATCH
                d_b = pltpu.async_copy(bi_h.at[pl.ds(base, BATCH)], biv, semi)
                d_s = pltpu.async_copy(si_h.at[pl.ds(base, BATCH)], siv, semi)
                d_d = pltpu.async_copy(di_h.at[pl.ds(base, BATCH)], div, semi)
                d_b.wait()
                d_s.wait()
                d_d.wait()
                for j in range(BATCH // 16):
                    sl = pl.ds(j * 16, 16)
                    vb = biv[sl]
                    ls = vb * N + siv[sl]
                    ld = vb * N + div[sl]
                    g0v[sl] = jnp.maximum(ls, 0)
                    g1v[sl] = jnp.maximum(ld, 0)
                    l0 = ld - lo
                    s0v[sl] = jnp.where((l0 < 0) | (l0 >= C), C, l0)
                    l1 = ls - lo
                    s1v[sl] = jnp.where((l1 < 0) | (l1 >= C), C, l1)
                g0 = pltpu.async_copy(xf_h.at[g0v], st0, sem0)
                g1 = pltpu.async_copy(xf_h.at[g1v], st1, sem1)
                g0.wait()
                pltpu.sync_copy(st0, acc.at[s0v], add=True)
                g1.wait()
                pltpu.sync_copy(st1, acc.at[s1v], add=True)
                return carry

            lax.fori_loop(0, n_batches, body, 0)
            plsc.subcore_barrier()
            pltpu.sync_copy(acc.at[pl.ds(s * wrows, wrows)],
                            out_h.at[pl.ds(lo + s * wrows, wrows)])
            plsc.subcore_barrier()

    return sc_call


def kernel(x, batch_idx, src, dst):
    B, N, D = x.shape
    R = B * N
    C = R // 4
    E = batch_idx.shape[0]
    per_tile = -(-E // (NS * BATCH)) * BATCH
    pad = per_tile * NS - E

    bi = jnp.concatenate([batch_idx.astype(jnp.int32),
                          jnp.full((pad,), -1, jnp.int32)])
    si = jnp.concatenate([src.astype(jnp.int32), jnp.zeros((pad,), jnp.int32)])
    di = jnp.concatenate([dst.astype(jnp.int32), jnp.zeros((pad,), jnp.int32)])
    xf = x.reshape(R, D)
    z = jnp.zeros((C + NS, D), jnp.float32)

    sc_call = _make_sc_call(R, D, C, per_tile, N)
    out_sum = sc_call(xf, bi, si, di, z)
    return jnp.concatenate([out_sum.reshape(B, N, D), x], axis=2)


# speedup vs baseline: 1.5315x; 1.5315x over previous
import functools

import jax
import jax.numpy as jnp
from jax import lax
from jax.experimental import pallas as pl
from jax.experimental.pallas import tpu as pltpu
from jax.experimental.pallas import tpu_sc as plsc

NC = 2
NS = 16
BATCH = 128


def _make_sc_call(R, D, C, per_tile, N):
    n_chunks_per_core = (R // C) // NC
    mesh = plsc.VectorSubcoreMesh(
        core_axis_name="c", subcore_axis_name="s",
        num_cores=NC, num_subcores=NS)
    acc_rows = -(-(C + 1) // (8 * NS)) * 8 * NS
    zrows = acc_rows // NS
    wrows = (C // NS) // 8 * 8
    wrem = C - wrows * NS

    @functools.partial(
        pl.kernel,
        out_type=jax.ShapeDtypeStruct((R, D), jnp.float32),
        mesh=mesh,
        scratch_types=[
            pltpu.VMEM_SHARED((acc_rows, 128), jnp.float32),
            pltpu.VMEM((BATCH,), jnp.int32),
            pltpu.VMEM((BATCH,), jnp.int32),
            pltpu.VMEM((BATCH,), jnp.int32),
            pltpu.VMEM((BATCH,), jnp.int32),
            pltpu.VMEM((BATCH,), jnp.int32),
            pltpu.VMEM((BATCH,), jnp.int32),
            pltpu.VMEM((BATCH,), jnp.int32),
            pltpu.VMEM((BATCH, 128), jnp.float32),
            pltpu.VMEM((BATCH, 128), jnp.float32),
            pltpu.SemaphoreType.DMA,
            pltpu.SemaphoreType.DMA,
            pltpu.SemaphoreType.DMA,
        ],
    )
    def sc_call(xf_h, bi_h, si_h, di_h, z_h, out_h,
                acc, biv, siv, div, g0v, g1v, s0v, s1v, st0, st1,
                semi, sem0, sem1):
        c = lax.axis_index("c")
        s = lax.axis_index("s")
        n_batches = per_tile // BATCH

        for p in range(n_chunks_per_core):
            lo = (n_chunks_per_core * c + p) * C
            pltpu.sync_copy(z_h.at[pl.ds(s * zrows, zrows)],
                            acc.at[pl.ds(s * zrows, zrows)])
            plsc.subcore_barrier()

            def body(i, carry):
                base = s * per_tile + i * BATCH
                d_b = pltpu.async_copy(bi_h.at[pl.ds(base, BATCH)], biv, semi)
                d_s = pltpu.async_copy(si_h.at[pl.ds(base, BATCH)], siv, semi)
                d_d = pltpu.async_copy(di_h.at[pl.ds(base, BATCH)], div, semi)
                d_b.wait()
                d_s.wait()
                d_d.wait()
                for j in range(BATCH // 16):
                    sl = pl.ds(j * 16, 16)
                    vb = biv[sl]
                    ls = vb * N + siv[sl]
                    ld = vb * N + div[sl]
                    g0v[sl] = jnp.maximum(ls, 0)
                    g1v[sl] = jnp.maximum(ld, 0)
                    l0 = ld - lo
                    s0v[sl] = jnp.where((l0 < 0) | (l0 >= C), C, l0)
                    l1 = ls - lo
                    s1v[sl] = jnp.where((l1 < 0) | (l1 >= C), C, l1)
                g0 = pltpu.async_copy(xf_h.at[g0v], st0, sem0)
                g1 = pltpu.async_copy(xf_h.at[g1v], st1, sem1)
                g0.wait()
                pltpu.sync_copy(st0, acc.at[s0v], add=True)
                g1.wait()
                pltpu.sync_copy(st1, acc.at[s1v], add=True)
                return carry

            lax.fori_loop(0, n_batches, body, 0)
            plsc.subcore_barrier()
            pltpu.sync_copy(acc.at[pl.ds(s * wrows, wrows)],
                            out_h.at[pl.ds(lo + s * wrows, wrows)])
            if wrem:
                @pl.when(s == 0)
                def _():
                    pltpu.sync_copy(acc.at[pl.ds(NS * wrows, wrem)],
                                    out_h.at[pl.ds(lo + NS * wrows, wrem)])
            plsc.subcore_barrier()

    return sc_call


def kernel(x, batch_idx, src, dst):
    B, N, D = x.shape
    R = B * N
    C = R // 4
    E = batch_idx.shape[0]
    per_tile = -(-E // (NS * BATCH)) * BATCH
    pad = per_tile * NS - E

    bi = jnp.concatenate([batch_idx.astype(jnp.int32),
                          jnp.full((pad,), -1, jnp.int32)])
    si = jnp.concatenate([src.astype(jnp.int32), jnp.zeros((pad,), jnp.int32)])
    di = jnp.concatenate([dst.astype(jnp.int32), jnp.zeros((pad,), jnp.int32)])
    xf = x.reshape(R, D)
    acc_rows = -(-(C + 1) // (8 * NS)) * 8 * NS
    z = jnp.zeros((acc_rows, D), jnp.float32)

    sc_call = _make_sc_call(R, D, C, per_tile, N)
    out_sum = sc_call(xf, bi, si, di, z)
    return jnp.concatenate([out_sum.reshape(B, N, D), x], axis=2)
